# Initial kernel scaffold; baseline (speedup 1.0000x reference)
import jax
import jax.numpy as jnp
from jax.experimental import pallas as pl


def kernel(X, M_x, A_x, label, W_ru, b_ru, W_c, b_c, W_out, b_out):
    raise NotImplementedError("write your pallas kernel here")


# speedup vs baseline: 24.0018x; 24.0018x over previous
import jax
import jax.numpy as jnp
from jax.experimental import pallas as pl

_B, _T, _N, _D = 2, 4, 2048, 128
_SQUARINGS = 13
_F32 = jnp.float32


def _dot(a, b):
    return jax.lax.dot_general(
        a, b, (((1,), (0,)), ((), ())), preferred_element_type=_F32
    )


def _dot_hi(a, b):
    return jax.lax.dot_general(
        a, b, (((1,), (0,)), ((), ())), preferred_element_type=_F32,
        precision=jax.lax.Precision.HIGHEST,
    )


def _sigma(W):
    G0 = jax.lax.dot_general(
        W, W, (((0,), (0,)), ((), ())), preferred_element_type=_F32,
        precision=jax.lax.Precision.HIGHEST,
    )
    d = G0.shape[0]
    rows = jax.lax.broadcasted_iota(jnp.int32, (d, d), 0)
    cols = jax.lax.broadcasted_iota(jnp.int32, (d, d), 1)
    eye = rows == cols
    zero = jnp.zeros((d, d), _F32)
    G = G0
    for _ in range(_SQUARINGS):
        t = jnp.sum(jnp.where(eye, G, zero))
        Gn = G / t
        G = _dot_hi(Gn, Gn)
    v = jnp.sum(G, axis=1, keepdims=True)
    Gv = jnp.sum(G0 * v.reshape(1, d), axis=1, keepdims=True)
    lam = jnp.sum(Gv * v) / jnp.sum(v * v)
    return jnp.sqrt(lam)


def _body(X_ref, A_ref, Wru_ref, bru_ref, Wc_ref, bc_ref, Wo_ref, bo_ref, out_ref):
    A = A_ref[...]
    bru = bru_ref[...]
    bc = bc_ref[...]
    wo = Wo_ref[...]

    Wru = Wru_ref[...] / (_sigma(Wru_ref[...]) + 1e-12)
    Wc = Wc_ref[...] / (_sigma(Wc_ref[...]) + 1e-12)
    won = wo / (jnp.sqrt(jnp.sum(wo * wo, dtype=_F32)) + 1e-12)

    Wru_d, Wru_a = Wru[0:2 * _D], Wru[2 * _D:4 * _D]
    Wc_d, Wc_a = Wc[0:2 * _D], Wc[2 * _D:4 * _D]
    Wru0 = jnp.concatenate([Wru[0:_D], Wru[2 * _D:3 * _D]], axis=0)
    Wc0 = jnp.concatenate([Wc[0:_D], Wc[2 * _D:3 * _D]], axis=0)

    hsum = jnp.zeros((1, _D), _F32)
    for b in range(_B):
        h = None
        for t in range(_T):
            x = X_ref[b, t]
            if h is None:
                Ax = _dot(A, x)
                xa = jnp.concatenate([x, Ax], axis=1)
                ru = jax.nn.sigmoid(_dot(xa, Wru0) + bru)
                u = ru[:, _D:2 * _D]
                c = jnp.tanh(_dot(xa, Wc0) + bc)
                h = (1.0 - u) * c
            else:
                inp = jnp.concatenate([x, h], axis=1)
                Ai = _dot(A, inp)
                ru = jax.nn.sigmoid(
                    _dot(jnp.concatenate([inp, Ai], axis=1), Wru) + bru)
                r, u = ru[:, 0:_D], ru[:, _D:2 * _D]
                rh = r * h
                Arh = _dot(A, rh)
                big = jnp.concatenate([x, rh, Ai[:, 0:_D], Arh], axis=1)
                c = jnp.tanh(_dot(big, Wc) + bc)
                h = u * h + (1.0 - u) * c
        hsum = hsum + jnp.sum(h, axis=0, keepdims=True)

    val = jnp.sum(hsum * won) / (_B * _N) + bo_ref[0, 0]
    out_ref[...] = jnp.reshape(val, (1, 1))


def kernel(X, M_x, A_x, label, W_ru, b_ru, W_c, b_c, W_out, b_out):
    del M_x, label
    out = pl.pallas_call(
        _body,
        out_shape=jax.ShapeDtypeStruct((1, 1), _F32),
    )(
        X,
        A_x,
        W_ru,
        b_ru.reshape(1, -1),
        W_c,
        b_c.reshape(1, -1),
        W_out.T,
        b_out.reshape(1, 1),
    )
    return out.reshape(())
